# baseline (speedup 1.0000x reference)
import jax
import jax.numpy as jnp
from jax import lax
from jax.experimental import pallas as pl

DIM = 192
E = 8
H = 64
W = 64
HW = H * W

_PW_ES = (0, 1, 2, 4, 5, 6)
_OFF_RW = 0
_OFF_PW = {e: 8 + i * DIM for i, e in enumerate(_PW_ES)}
_OFF_DW0 = 8 + 6 * DIM
_OFF_DW4 = _OFF_DW0 + 16
_OFF_F3A = _OFF_DW4 + 16
_OFF_F3B = _OFF_F3A + 48
_OFF_F7A = _OFF_F3B + 48
_OFF_F7B = _OFF_F7A + 48
_S0 = _OFF_F7B + 48
_OFF_RB = _S0 + 0
_OFF_EB = _S0 + 1
_OFF_PB = {e: _S0 + 2 + i for i, e in enumerate(_PW_ES)}
_OFF_G1 = _S0 + 8
_OFF_G5 = _S0 + 9
_OFF_DB0 = _S0 + 10
_OFF_DB4 = _S0 + 11
_OFF_F3AB = _S0 + 12
_OFF_F3BB = _S0 + 13
_OFF_F7AB = _S0 + 14
_OFF_F7BB = _S0 + 15
_N_ROWS = _S0 + 16


def _shift(a, dh, dw):
    if dh > 0:
        a = jnp.concatenate([a[dh:], jnp.zeros((dh,) + a.shape[1:], a.dtype)], axis=0)
    elif dh < 0:
        a = jnp.concatenate([jnp.zeros((-dh,) + a.shape[1:], a.dtype), a[:dh]], axis=0)
    if dw > 0:
        a = jnp.concatenate([a[:, dw:], jnp.zeros(a.shape[:1] + (dw,) + a.shape[2:], a.dtype)], axis=1)
    elif dw < 0:
        a = jnp.concatenate([jnp.zeros(a.shape[:1] + (-dw,) + a.shape[2:], a.dtype), a[:, :dw]], axis=1)
    return a


def _matmul_ct(a, w):
    return lax.dot_general(a, w, (((1,), (1,)), ((), ())),
                           preferred_element_type=jnp.float32)


def _matmul_nt(a, w):
    return lax.dot_general(a, w, (((1,), (0,)), ((), ())),
                           preferred_element_type=jnp.float32)


def _moe_step(x_ref, wt_ref, out_ref):
    xb = x_ref[0]
    xf = xb.reshape(HW, DIM)

    def rows(off, n):
        return wt_ref[off:off + n, :]

    def row1(off, n=DIM):
        return wt_ref[off:off + 1, :n]

    gvec = jnp.mean(xf, axis=0, keepdims=True)
    logits = _matmul_ct(gvec, rows(_OFF_RW, E)) + row1(_OFF_RB, E)
    logits = jnp.clip(logits, -10.0, 10.0) + row1(_OFF_EB, E)
    m = jnp.max(logits)
    p = jnp.exp(logits - m)
    probs = p / jnp.sum(p)
    probs = jnp.clip(probs, 1e-6, 1.0)

    iota = lax.broadcasted_iota(jnp.int32, (1, E), 1)
    v1 = jnp.max(probs)
    i1 = jnp.min(jnp.where(probs == v1, iota, E))
    sel1 = iota == i1
    rest = jnp.where(sel1, -jnp.inf, probs)
    v2 = jnp.max(rest)
    i2 = jnp.min(jnp.where((rest == v2) & (~sel1), iota, E))
    sel2 = iota == i2
    denom = v1 + v2 + 1e-8
    wa = v1 / denom
    wb = v2 / denom
    gates = jnp.where(sel1, wa, 0.0) + jnp.where(sel2, wb, 0.0)

    def gate(e):
        return jnp.sum(jnp.where(iota == e, gates, 0.0))

    g0, g1, g2, g3 = gate(0), gate(1), gate(2), gate(3)
    g4, g5, g6, g7 = gate(4), gate(5), gate(6), gate(7)

    def s_vec(fa, fab, fb, fbb):
        h = jnp.maximum(_matmul_ct(gvec, rows(fa, 48)) + row1(fab, 48), 0.0)
        return jax.nn.sigmoid(_matmul_nt(h, rows(fb, 48)) + row1(fbb))

    alpha = ((wa + wb)
             + g3 * s_vec(_OFF_F3A, _OFF_F3AB, _OFF_F3B, _OFF_F3BB)
             + g7 * s_vec(_OFF_F7A, _OFF_F7AB, _OFF_F7B, _OFF_F7BB))

    btot = (g0 * row1(_OFF_PB[0]) + g1 * row1(_OFF_PB[1]) + g2 * row1(_OFF_PB[2])
            + g4 * row1(_OFF_PB[4]) + g5 * row1(_OFF_PB[5]) + g6 * row1(_OFF_PB[6]))

    out_ref[0] = (xf * alpha + btot).reshape(H, W, DIM)

    @pl.when(g1 + g5 > 0.0)
    def _freq():
        wfr = (g1 * (rows(_OFF_PW[1], DIM) * row1(_OFF_G1))
               + g5 * (rows(_OFF_PW[5], DIM) * row1(_OFF_G5)))
        out_ref[0] += _matmul_ct(xf, wfr).reshape(H, W, DIM)

    @pl.when(g2 + g6 > 0.0)
    def _edge():
        lap = (_shift(xb, -1, 0) + _shift(xb, 1, 0) +
               _shift(xb, 0, -1) + _shift(xb, 0, 1) - 4.0 * xb)
        wed = g2 * rows(_OFF_PW[2], DIM) + g6 * rows(_OFF_PW[6], DIM)
        out_ref[0] += _matmul_ct(lap.reshape(HW, DIM), wed).reshape(H, W, DIM)

    def texture(dw_off, db_off, w_off, g):
        acc = jnp.broadcast_to(row1(db_off)[None], (H, W, DIM))
        for a in range(3):
            for c in range(3):
                acc = acc + _shift(xb, a - 1, c - 1) * wt_ref[dw_off + a * 3 + c][None, None, :]
        u = jax.nn.gelu(acc)
        out_ref[0] += _matmul_ct(u.reshape(HW, DIM), g * rows(w_off, DIM)).reshape(H, W, DIM)

    @pl.when(g0 > 0.0)
    def _tex0():
        texture(_OFF_DW0, _OFF_DB0, _OFF_PW[0], g0)

    @pl.when(g4 > 0.0)
    def _tex4():
        texture(_OFF_DW4, _OFF_DB4, _OFF_PW[4], g4)


def kernel(x, params):
    B = x.shape[0]
    xh = jnp.transpose(x, (0, 2, 3, 1))

    def padrow(v):
        return jnp.pad(v[None, :], ((0, 0), (0, DIM - v.shape[0])))

    def row(v):
        return v[None, :]

    zpad7 = jnp.zeros((7, DIM), jnp.float32)
    pieces = [params['router_w']]
    for e in _PW_ES:
        pieces.append(params[f'e{e}_pw_w'].reshape(DIM, DIM))
    pieces += [params['e0_dw_w'].reshape(DIM, 9).T, zpad7,
               params['e4_dw_w'].reshape(DIM, 9).T, zpad7,
               params['e3_fc1_w'], params['e3_fc2_w'].T,
               params['e7_fc1_w'], params['e7_fc2_w'].T,
               padrow(params['router_b']), padrow(params['expert_bias'])]
    for e in _PW_ES:
        pieces.append(row(params[f'e{e}_pw_b']))
    pieces += [row(params['e1_gain']), row(params['e5_gain']),
               row(params['e0_dw_b']), row(params['e4_dw_b']),
               padrow(params['e3_fc1_b']), row(params['e3_fc2_b']),
               padrow(params['e7_fc1_b']), row(params['e7_fc2_b'])]
    wt = jnp.concatenate(pieces, axis=0)
    assert wt.shape == (_N_ROWS, DIM), wt.shape

    out_h = pl.pallas_call(
        _moe_step,
        grid=(B,),
        in_specs=[
            pl.BlockSpec((1, H, W, DIM), lambda b: (b, 0, 0, 0)),
            pl.BlockSpec((_N_ROWS, DIM), lambda b: (0, 0)),
        ],
        out_specs=pl.BlockSpec((1, H, W, DIM), lambda b: (b, 0, 0, 0)),
        out_shape=jax.ShapeDtypeStruct((B, H, W, DIM), jnp.float32),
    )(xh, wt)

    out = jnp.transpose(out_h, (0, 3, 1, 2))
    return (out, jnp.array(0.0, dtype=x.dtype))
